# baseline (speedup 1.0000x reference)
import functools

import jax
import jax.numpy as jnp
from jax import lax
from jax.experimental import pallas as pl
from jax.experimental.pallas import tpu as pltpu
from jax.experimental.pallas import tpu_sc as plsc

_NUM_CORES = 2
_NUM_SUBCORES = 16
_CH = 128
_NPASS = 2


def _pad_nodes(n):
    return -(-n // (8 * _NUM_SUBCORES)) * 8 * _NUM_SUBCORES


def _prep_indices(src, dst, e, workers, trash_row):
    per = -(-e // (workers * _CH * _NPASS))
    per += per % 2
    pad = workers * _NPASS * per * _CH - e

    def with_pad(idx, fill):
        filler = jnp.full((pad,), fill, jnp.int32)
        p = jnp.concatenate([idx, filler]).reshape(workers, _NPASS, per, _CH)
        prime = jnp.full((workers, _NPASS, 2, _CH), fill, jnp.int32)
        return jnp.concatenate([p, prime], axis=2)

    return (with_pad(src, 0), with_pad(dst, trash_row), per)


def _make_sc_agg(n, d, nch):
    n_pad = _pad_nodes(n)
    rps = n_pad // _NUM_SUBCORES
    mesh = plsc.VectorSubcoreMesh(core_axis_name="c", subcore_axis_name="s")

    @functools.partial(
        pl.kernel,
        out_type=[jax.ShapeDtypeStruct((n_pad, d), jnp.float32),
                  jax.ShapeDtypeStruct((n_pad, d), jnp.float32)],
        mesh=mesh,
        scratch_types=[
            pltpu.VMEM_SHARED((n_pad, d), jnp.float32),
            pltpu.VMEM((nch + 2, _CH), jnp.int32),
            pltpu.VMEM((nch + 2, _CH), jnp.int32),
            pltpu.VMEM((_CH, d), jnp.float32),
            pltpu.VMEM((_CH, d), jnp.float32),
            pltpu.SemaphoreType.DMA,
            pltpu.SemaphoreType.DMA,
        ],
    )
    def k(feat_hbm, zeros_hbm, src_hbm, dst_hbm, out0_hbm, out1_hbm,
          acc, sidx, didx, rows0, rows1, s0, s1):
        cid = lax.axis_index("c")
        sid = lax.axis_index("s")
        wid = cid * _NUM_SUBCORES + sid
        pltpu.sync_copy(zeros_hbm.at[pl.ds(sid * rps, rps)],
                        acc.at[pl.ds(sid * rps, rps)])
        plsc.subcore_barrier()

        def g_start(j, rows, sem):
            pltpu.make_async_copy(feat_hbm.at[sidx.at[j]], rows, sem).start()

        def g_wait(j, rows, sem):
            pltpu.make_async_copy(feat_hbm.at[sidx.at[j]], rows, sem).wait()

        def scat(j, rows):
            pltpu.sync_copy(rows, acc.at[didx.at[j]], add=True)

        for p in range(_NPASS):
            pltpu.sync_copy(src_hbm.at[wid, p], sidx)
            pltpu.sync_copy(dst_hbm.at[wid, p], didx)

            g_start(0, rows0, s0)
            g_start(1, rows1, s1)

            @pl.loop(0, nch, step=2)
            def _(j):
                g_wait(j, rows0, s0)
                scat(j, rows0)
                g_start(j + 2, rows0, s0)
                g_wait(j + 1, rows1, s1)
                scat(j + 1, rows1)
                g_start(j + 3, rows1, s1)

            g_wait(nch, rows0, s0)
            g_wait(nch + 1, rows1, s1)

        plsc.subcore_barrier()

        @pl.when(cid == 0)
        def _():
            pltpu.sync_copy(acc.at[pl.ds(sid * rps, rps)],
                            out0_hbm.at[pl.ds(sid * rps, rps)])

        @pl.when(cid == 1)
        def _():
            pltpu.sync_copy(acc.at[pl.ds(sid * rps, rps)],
                            out1_hbm.at[pl.ds(sid * rps, rps)])

    return k


def _tc_layer1(x, p0, p1, w, b):
    n, d_in = x.shape
    d_out = w.shape[1]
    dh = d_out // 2
    br = 1000
    grid = (n // br,)

    def body(x_ref, p0_ref, p1_ref, w_ref, b_ref, o0_ref, o1_ref):
        h = x_ref[...] + p0_ref[...] + p1_ref[...]
        y = lax.dot_general(h, w_ref[...], (((1,), (0,)), ((), ())),
                            precision=lax.Precision.HIGHEST,
                            preferred_element_type=jnp.float32)
        y = jnp.maximum(y + b_ref[...], 0.0)
        o0_ref[...] = y[:, :dh]
        o1_ref[...] = y[:, dh:]

    return pl.pallas_call(
        body,
        grid=grid,
        in_specs=[
            pl.BlockSpec((br, d_in), lambda i: (i, 0)),
            pl.BlockSpec((br, d_in), lambda i: (i, 0)),
            pl.BlockSpec((br, d_in), lambda i: (i, 0)),
            pl.BlockSpec((d_in, d_out), lambda i: (0, 0)),
            pl.BlockSpec((1, d_out), lambda i: (0, 0)),
        ],
        out_specs=[
            pl.BlockSpec((br, dh), lambda i: (i, 0)),
            pl.BlockSpec((br, dh), lambda i: (i, 0)),
        ],
        out_shape=[jax.ShapeDtypeStruct((n, dh), jnp.float32),
                   jax.ShapeDtypeStruct((n, dh), jnp.float32)],
    )(x, p0, p1, w, b.reshape(1, d_out))


def _tc_layer2(h0, h1, q0a, q0b, q1a, q1b, w, b):
    n, dh = h0.shape
    d_out = w.shape[1]
    br = 1000
    grid = (n // br,)

    def body(h0_ref, h1_ref, a_ref, b2_ref, c_ref, d_ref, w_ref, bias_ref,
             o_ref):
        h = jnp.concatenate(
            [h0_ref[...] + a_ref[...] + b2_ref[...],
             h1_ref[...] + c_ref[...] + d_ref[...]], axis=1)
        y = lax.dot_general(h, w_ref[...], (((1,), (0,)), ((), ())),
                            precision=lax.Precision.HIGHEST,
                            preferred_element_type=jnp.float32)
        o_ref[...] = y + bias_ref[...]

    row_spec = pl.BlockSpec((br, dh), lambda i: (i, 0))
    return pl.pallas_call(
        body,
        grid=grid,
        in_specs=[
            row_spec, row_spec, row_spec, row_spec, row_spec, row_spec,
            pl.BlockSpec((2 * dh, d_out), lambda i: (0, 0)),
            pl.BlockSpec((1, d_out), lambda i: (0, 0)),
        ],
        out_specs=pl.BlockSpec((br, d_out), lambda i: (i, 0)),
        out_shape=jax.ShapeDtypeStruct((n, d_out), jnp.float32),
    )(h0, h1, q0a, q0b, q1a, q1b, w, b.reshape(1, d_out))


def kernel(x, edge_index, W1, b1, W2, b2):
    n, d_in = x.shape
    e = edge_index.shape[1]
    d_hid = W1.shape[1]
    dh = d_hid // 2

    src = edge_index[0].astype(jnp.int32)
    dst = edge_index[1].astype(jnp.int32)

    n_pad = _pad_nodes(n)
    zeros = jnp.zeros((n_pad, d_in), jnp.float32)

    w_all = _NUM_CORES * _NUM_SUBCORES
    src3, dst3, nch = _prep_indices(src, dst, e, w_all, n)

    sc = _make_sc_agg(n, d_in, nch)
    p0, p1 = sc(x, zeros, src3, dst3)
    h0, h1 = _tc_layer1(x, p0, p1, W1, b1)

    q0a, q0b = sc(h0, zeros, src3, dst3)
    q1a, q1b = sc(h1, zeros, src3, dst3)
    return _tc_layer2(h0, h1, q0a, q0b, q1a, q1b, W2, b2)


# speedup vs baseline: 2.0790x; 2.0790x over previous
import functools

import jax
import jax.numpy as jnp
from jax import lax
from jax.experimental import pallas as pl
from jax.experimental.pallas import tpu as pltpu
from jax.experimental.pallas import tpu_sc as plsc

_NUM_CORES = 2
_NUM_SUBCORES = 16
_CH = 128
_NPASS = 2


def _pad_nodes(n):
    return -(-n // (8 * _NUM_SUBCORES)) * 8 * _NUM_SUBCORES


def _prep_indices(src, dst, e, workers, trash_row):
    per = -(-e // (workers * _CH * _NPASS))
    per += per % 2
    pad = workers * _NPASS * per * _CH - e

    def with_pad(idx, fill):
        filler = jnp.full((pad,), fill, jnp.int32)
        p = jnp.concatenate([idx, filler]).reshape(workers, _NPASS, per, _CH)
        prime = jnp.full((workers, _NPASS, 2, _CH), fill, jnp.int32)
        return jnp.concatenate([p, prime], axis=2)

    return (with_pad(src, 0), with_pad(dst, trash_row), per)


def _make_sc_agg(n, d, nch):
    n_pad = _pad_nodes(n)
    rps = n_pad // _NUM_SUBCORES
    mesh = plsc.VectorSubcoreMesh(core_axis_name="c", subcore_axis_name="s")

    @functools.partial(
        pl.kernel,
        out_type=[jax.ShapeDtypeStruct((n_pad, d), jnp.float32),
                  jax.ShapeDtypeStruct((n_pad, d), jnp.float32)],
        mesh=mesh,
        scratch_types=[
            pltpu.VMEM_SHARED((n_pad, d), jnp.float32),
            pltpu.VMEM((nch + 2, _CH), jnp.int32),
            pltpu.VMEM((nch + 2, _CH), jnp.int32),
            pltpu.VMEM((_CH, d), jnp.float32),
            pltpu.VMEM((_CH, d), jnp.float32),
            pltpu.SemaphoreType.DMA,
            pltpu.SemaphoreType.DMA,
        ],
    )
    def k(feat_hbm, zeros_hbm, src_hbm, dst_hbm, out0_hbm, out1_hbm,
          acc, sidx, didx, rows0, rows1, s0, s1):
        cid = lax.axis_index("c")
        sid = lax.axis_index("s")
        wid = cid * _NUM_SUBCORES + sid
        pltpu.sync_copy(zeros_hbm.at[pl.ds(sid * rps, rps)],
                        acc.at[pl.ds(sid * rps, rps)])
        plsc.subcore_barrier()

        def g_start(j, rows, sem):
            pltpu.make_async_copy(feat_hbm.at[sidx.at[j]], rows, sem).start()

        def g_wait(j, rows, sem):
            pltpu.make_async_copy(feat_hbm.at[sidx.at[j]], rows, sem).wait()

        def scat(j, rows):
            pltpu.sync_copy(rows, acc.at[didx.at[j]], add=True)

        for p in range(_NPASS):
            pltpu.sync_copy(src_hbm.at[wid, p], sidx)
            pltpu.sync_copy(dst_hbm.at[wid, p], didx)

            @pl.loop(0, nch)
            def _(j):
                pltpu.sync_copy(feat_hbm.at[sidx.at[j]], rows0)
                scat(j, rows0)

        plsc.subcore_barrier()

        @pl.when(cid == 0)
        def _():
            pltpu.sync_copy(acc.at[pl.ds(sid * rps, rps)],
                            out0_hbm.at[pl.ds(sid * rps, rps)])

        @pl.when(cid == 1)
        def _():
            pltpu.sync_copy(acc.at[pl.ds(sid * rps, rps)],
                            out1_hbm.at[pl.ds(sid * rps, rps)])

    return k


def _tc_layer1(x, p0, p1, w, b):
    n, d_in = x.shape
    d_out = w.shape[1]
    dh = d_out // 2
    br = 1000
    grid = (n // br,)

    def body(x_ref, p0_ref, p1_ref, w_ref, b_ref, o0_ref, o1_ref):
        h = x_ref[...] + p0_ref[...] + p1_ref[...]
        y = lax.dot_general(h, w_ref[...], (((1,), (0,)), ((), ())),
                            precision=lax.Precision.HIGHEST,
                            preferred_element_type=jnp.float32)
        y = jnp.maximum(y + b_ref[...], 0.0)
        o0_ref[...] = y[:, :dh]
        o1_ref[...] = y[:, dh:]

    return pl.pallas_call(
        body,
        grid=grid,
        in_specs=[
            pl.BlockSpec((br, d_in), lambda i: (i, 0)),
            pl.BlockSpec((br, d_in), lambda i: (i, 0)),
            pl.BlockSpec((br, d_in), lambda i: (i, 0)),
            pl.BlockSpec((d_in, d_out), lambda i: (0, 0)),
            pl.BlockSpec((1, d_out), lambda i: (0, 0)),
        ],
        out_specs=[
            pl.BlockSpec((br, dh), lambda i: (i, 0)),
            pl.BlockSpec((br, dh), lambda i: (i, 0)),
        ],
        out_shape=[jax.ShapeDtypeStruct((n, dh), jnp.float32),
                   jax.ShapeDtypeStruct((n, dh), jnp.float32)],
    )(x, p0, p1, w, b.reshape(1, d_out))


def _tc_layer2(h0, h1, q0a, q0b, q1a, q1b, w, b):
    n, dh = h0.shape
    d_out = w.shape[1]
    br = 1000
    grid = (n // br,)

    def body(h0_ref, h1_ref, a_ref, b2_ref, c_ref, d_ref, w_ref, bias_ref,
             o_ref):
        h = jnp.concatenate(
            [h0_ref[...] + a_ref[...] + b2_ref[...],
             h1_ref[...] + c_ref[...] + d_ref[...]], axis=1)
        y = lax.dot_general(h, w_ref[...], (((1,), (0,)), ((), ())),
                            precision=lax.Precision.HIGHEST,
                            preferred_element_type=jnp.float32)
        o_ref[...] = y + bias_ref[...]

    row_spec = pl.BlockSpec((br, dh), lambda i: (i, 0))
    return pl.pallas_call(
        body,
        grid=grid,
        in_specs=[
            row_spec, row_spec, row_spec, row_spec, row_spec, row_spec,
            pl.BlockSpec((2 * dh, d_out), lambda i: (0, 0)),
            pl.BlockSpec((1, d_out), lambda i: (0, 0)),
        ],
        out_specs=pl.BlockSpec((br, d_out), lambda i: (i, 0)),
        out_shape=jax.ShapeDtypeStruct((n, d_out), jnp.float32),
    )(h0, h1, q0a, q0b, q1a, q1b, w, b.reshape(1, d_out))


def kernel(x, edge_index, W1, b1, W2, b2):
    n, d_in = x.shape
    e = edge_index.shape[1]
    d_hid = W1.shape[1]
    dh = d_hid // 2

    src = edge_index[0].astype(jnp.int32)
    dst = edge_index[1].astype(jnp.int32)

    n_pad = _pad_nodes(n)
    zeros = jnp.zeros((n_pad, d_in), jnp.float32)

    w_all = _NUM_CORES * _NUM_SUBCORES
    src3, dst3, nch = _prep_indices(src, dst, e, w_all, n)

    sc = _make_sc_agg(n, d_in, nch)
    p0, p1 = sc(x, zeros, src3, dst3)
    h0, h1 = _tc_layer1(x, p0, p1, W1, b1)

    q0a, q0b = sc(h0, zeros, src3, dst3)
    q1a, q1b = sc(h1, zeros, src3, dst3)
    return _tc_layer2(h0, h1, q0a, q0b, q1a, q1b, W2, b2)
